# Initial kernel scaffold; baseline (speedup 1.0000x reference)
import jax
import jax.numpy as jnp
from jax.experimental import pallas as pl


def kernel(out_hm, out_hm_kpts, out_kpts_center_offset, out_reg, out_w, out_kpts_offset, out_scales, gt_hm, gt_hm_kpts, ind, kpts_ind, b_kpts_center_offset, b_kpts_center_mask, b_reg, b_reg_mask, b_w, b_w_mask, b_kpts_offset, b_kpts_mask, b_scales, b_scales_mask):
    raise NotImplementedError("write your pallas kernel here")


# speedup vs baseline: 1.1788x; 1.1788x over previous
import numpy as np
import jax
import jax.numpy as jnp
from jax import lax
from jax.experimental import pallas as pl
from jax.experimental.pallas import tpu as pltpu

_LOG_LO = float(np.log(1e-4))
_LOG_HI = float(np.log(1.0 - 1e-4))


def _fused_kernel(hmx, hmg, kpx, kpg,
                  ind, kind,
                  mkc, mrg, mw, mko, msc,
                  tkc, trg, tw, tko, tsc,
                  fkc, frg, fw, fko, fsc,
                  focal_out, reg_out):
    r = pl.program_id(1)

    @pl.when(r == 0)
    def _():
        focal_out[...] = jnp.zeros_like(focal_out)
        reg_out[...] = jnp.zeros_like(reg_out)

    def focal_partials(x_ref, gt_ref):
        x = x_ref[...]
        gt = gt_ref[...]
        e = jnp.exp(-jnp.abs(x))
        lp = jnp.where(x >= 0.0, 0.0, x) - jnp.log1p(e)
        lpc = jnp.clip(lp, _LOG_LO, _LOG_HI)
        lqc = jnp.clip(lp - x, _LOG_LO, _LOG_HI)
        sig = jnp.where(x >= 0.0, 1.0, e) / (1.0 + e)
        pred = jnp.clip(sig, 1e-4, 1.0 - 1e-4)
        one_m = 1.0 - pred

        pos_inds = (gt == 1.0).astype(jnp.float32)
        neg_inds = (gt < 1.0).astype(jnp.float32)
        neg_w = (1.0 - gt) ** 4

        ppos = jnp.sum(lpc * one_m * one_m * pos_inds, axis=0, keepdims=True)
        pneg = jnp.sum(lqc * pred * pred * neg_w * neg_inds, axis=0,
                       keepdims=True)
        pnum = jnp.sum(pos_inds, axis=0, keepdims=True)
        return ppos, pneg, pnum

    p1, n1, c1 = focal_partials(hmx, hmg)
    p2, n2, c2 = focal_partials(kpx, kpg)
    upd = jnp.concatenate([p1, n1, c1, p2, n2, c2], axis=0)
    focal_out[0] = focal_out[0] + upd

    h_dim = fkc.shape[2]
    w_dim = fkc.shape[3]
    k_n = ind.shape[1]
    iv = ind[0]
    kv = kind[0]
    lane_h = lax.broadcasted_iota(jnp.int32, (k_n, h_dim), 1)
    lane_w = lax.broadcasted_iota(jnp.int32, (k_n, w_dim), 1)
    oh_h = (lane_h == iv // w_dim).astype(jnp.float32)
    oh_w = (lane_w == iv % w_dim).astype(jnp.float32)
    oh_hk = (lane_h == kv // w_dim).astype(jnp.float32)
    oh_wk = (lane_w == kv % w_dim).astype(jnp.float32)

    vals = []
    for f, m, t, ohh, ohw in ((fkc, mkc, tkc, oh_h, oh_w),
                              (frg, mrg, trg, oh_h, oh_w),
                              (fw, mw, tw, oh_h, oh_w),
                              (fko, mko, tko, oh_hk, oh_wk),
                              (fsc, msc, tsc, oh_h, oh_w)):
        mm = m[0]
        tt = t[0]
        lsum = 0.0
        for ci in range(f.shape[1]):
            g = jnp.dot(ohh, f[0, ci],
                        preferred_element_type=jnp.float32)
            pred = jnp.sum(g * ohw, axis=1, keepdims=True)
            contrib = jnp.abs((pred - tt[:, ci:ci + 1]) * mm[:, ci:ci + 1])
            lsum = lsum + jnp.sum(contrib)
        vals.append(lsum)
        vals.append(jnp.sum(mm))

    lane16 = lax.broadcasted_iota(jnp.int32, (1, 16), 1)
    row = jnp.zeros((1, 16), jnp.float32)
    for j, v in enumerate(vals):
        row = row + jnp.where(lane16 == j, v, 0.0)
    reg_out[0] = reg_out[0] + row


def kernel(out_hm, out_hm_kpts, out_kpts_center_offset, out_reg, out_w,
           out_kpts_offset, out_scales, gt_hm, gt_hm_kpts, ind, kpts_ind,
           b_kpts_center_offset, b_kpts_center_mask, b_reg, b_reg_mask,
           b_w, b_w_mask, b_kpts_offset, b_kpts_mask, b_scales, b_scales_mask):
    B, C_hm, H, W = out_hm.shape
    nb = B // 2
    br = C_hm * H

    hmx = jnp.reshape(out_hm.astype(jnp.float32), (B * br, W))
    hmg = jnp.reshape(gt_hm.astype(jnp.float32), (B * br, W))
    kpx = jnp.reshape(out_hm_kpts.astype(jnp.float32), (B * br, W))
    kpg = jnp.reshape(gt_hm_kpts.astype(jnp.float32), (B * br, W))

    feats = [out_kpts_center_offset.astype(jnp.float32),
             out_reg.astype(jnp.float32),
             out_w.astype(jnp.float32),
             out_kpts_offset.astype(jnp.float32),
             out_scales.astype(jnp.float32)]
    tgts = [b_kpts_center_offset.astype(jnp.float32),
            b_reg.astype(jnp.float32),
            b_w.astype(jnp.float32),
            b_kpts_offset.astype(jnp.float32),
            b_scales.astype(jnp.float32)]
    masks = []
    for mk, tg in zip((b_kpts_center_mask, b_reg_mask, b_w_mask,
                       b_kpts_mask, b_scales_mask), tgts):
        mk = mk.astype(jnp.float32)
        if mk.ndim == 2:
            mk = jnp.broadcast_to(mk[:, :, None], tg.shape)
        masks.append(mk)

    K = ind.shape[1]
    ind3 = jnp.reshape(ind.astype(jnp.int32), (B, K, 1))
    kind3 = jnp.reshape(kpts_ind.astype(jnp.int32), (B, kpts_ind.shape[1], 1))

    fmap = lambda c, r: (c * nb + r, 0)
    bmap = lambda c, r: (c * nb + r, 0, 0)
    f4map = lambda c, r: (c * nb + r, 0, 0, 0)

    focal_specs = [pl.BlockSpec((br, W), fmap)] * 4
    ind_specs = [pl.BlockSpec((1, K, 1), bmap),
                 pl.BlockSpec((1, kpts_ind.shape[1], 1), bmap)]
    mt_specs = [pl.BlockSpec((1,) + m.shape[1:], bmap) for m in masks]
    mt_specs += [pl.BlockSpec((1,) + t.shape[1:], bmap) for t in tgts]
    feat_specs = [pl.BlockSpec((1,) + f.shape[1:], f4map) for f in feats]

    focal_out, reg_out = pl.pallas_call(
        _fused_kernel,
        out_shape=[jax.ShapeDtypeStruct((2, 6, W), jnp.float32),
                   jax.ShapeDtypeStruct((2, 1, 16), jnp.float32)],
        grid=(2, nb),
        in_specs=focal_specs + ind_specs + mt_specs + feat_specs,
        out_specs=[pl.BlockSpec((1, 6, W), lambda c, r: (c, 0, 0)),
                   pl.BlockSpec((1, 1, 16), lambda c, r: (c, 0, 0))],
        compiler_params=pltpu.CompilerParams(
            dimension_semantics=("parallel", "arbitrary"),
            vmem_limit_bytes=64 * 1024 * 1024),
    )(hmx, hmg, kpx, kpg, ind3, kind3, *masks, *tgts, *feats)

    fsum = jnp.sum(focal_out, axis=(0, 2))

    def _floss(pos, neg, npos):
        return jnp.where(npos == 0, -neg,
                         -(pos + neg) / jnp.maximum(npos, 1.0))

    hm_loss = _floss(fsum[0], fsum[1], fsum[2])
    hm_kpts_loss = _floss(fsum[3], fsum[4], fsum[5])

    rs = jnp.reshape(jnp.sum(reg_out, axis=0), (-1,))
    kpts_center_loss = rs[0] / (rs[1] + 1e-4)
    off_loss = rs[2] / (rs[3] + 1e-4)
    w_loss = rs[4] / (rs[5] + 1e-4)
    kpts_offset_loss = rs[6] / (rs[7] + 1e-4)
    scale_loss = rs[8] / (rs[9] + 1e-4)

    loss = (hm_loss + 0.1 * w_loss + off_loss + kpts_center_loss
            + hm_kpts_loss + kpts_offset_loss + scale_loss)
    loss_stats = {'loss': loss, 'hm_loss': hm_loss, 'w_loss': w_loss,
                  'kpts_center_loss': kpts_center_loss,
                  'reg_loss(center_offset)': off_loss,
                  'hm_kpts_loss': hm_kpts_loss,
                  'kpts_offset_loss': kpts_offset_loss,
                  'scale_loss': scale_loss}
    return loss, loss_stats
